# Initial kernel scaffold; baseline (speedup 1.0000x reference)
import jax
import jax.numpy as jnp
from jax.experimental import pallas as pl


def kernel(x):
    raise NotImplementedError("write your pallas kernel here")


# speedup vs baseline: 157.5237x; 157.5237x over previous
import functools

import jax
import jax.numpy as jnp
from jax import lax
from jax.experimental import pallas as pl
from jax.experimental.pallas import tpu as pltpu
from jax.experimental.pallas import tpu_sc as plsc

_NUM_BUCKETS = 16


def kernel(x):
    orig_shape = x.shape
    n = x.size
    info = plsc.get_sparse_core_info()
    num_cores, num_subcores, lanes = info.num_cores, info.num_subcores, info.num_lanes
    num_workers = num_cores * num_subcores
    assert n % (num_workers * lanes) == 0
    chunk = n // num_workers

    buckets = jnp.linspace(-1.0, 1.0, _NUM_BUCKETS).astype(jnp.float32)
    x_flat = x.reshape(n)

    mesh = plsc.VectorSubcoreMesh(core_axis_name="c", subcore_axis_name="s")

    @functools.partial(
        pl.kernel,
        mesh=mesh,
        out_type=(
            jax.ShapeDtypeStruct((n,), jnp.int32),
            jax.ShapeDtypeStruct((n,), jnp.float32),
        ),
        scratch_types=[
            pltpu.VMEM((chunk,), jnp.float32),
            pltpu.VMEM((chunk,), jnp.int32),
            pltpu.VMEM((chunk,), jnp.float32),
            pltpu.VMEM((_NUM_BUCKETS,), jnp.float32),
        ],
    )
    def _quantize(x_hbm, b_hbm, idx_hbm, val_hbm, x_v, idx_v, val_v, b_v):
        wid = lax.axis_index("s") * num_cores + lax.axis_index("c")
        base = wid * chunk
        pltpu.sync_copy(b_hbm, b_v)
        pltpu.sync_copy(x_hbm.at[pl.ds(base, chunk)], x_v)
        b_vec = b_v[...]

        def body(i, carry):
            off = i * lanes
            v = x_v[pl.ds(off, lanes)]
            v = jnp.minimum(jnp.maximum(v, -1.0), 1.0)
            t = v * 7.5 + 8.0
            q = t.astype(jnp.int32)
            idx_v[pl.ds(off, lanes)] = q
            val_v[pl.ds(off, lanes)] = lax.gather(
                b_vec,
                q[:, None],
                dimension_numbers=lax.GatherDimensionNumbers(
                    offset_dims=(), collapsed_slice_dims=(0,),
                    start_index_map=(0,)),
                slice_sizes=(1,),
                mode=lax.GatherScatterMode.PROMISE_IN_BOUNDS,
            )
            return carry

        lax.fori_loop(0, chunk // lanes, body, 0)

        pltpu.sync_copy(idx_v, idx_hbm.at[pl.ds(base, chunk)])
        pltpu.sync_copy(val_v, val_hbm.at[pl.ds(base, chunk)])

    idx, vals = _quantize(x_flat, buckets)
    return idx.reshape(orig_shape), vals.reshape(orig_shape)


# speedup vs baseline: 167.5757x; 1.0638x over previous
import functools

import jax
import jax.numpy as jnp
from jax import lax
from jax.experimental import pallas as pl
from jax.experimental.pallas import tpu as pltpu
from jax.experimental.pallas import tpu_sc as plsc

_NUM_BUCKETS = 16


def kernel(x):
    orig_shape = x.shape
    n = x.size
    info = plsc.get_sparse_core_info()
    num_cores, num_subcores, lanes = info.num_cores, info.num_subcores, info.num_lanes
    num_workers = num_cores * num_subcores
    assert n % (num_workers * lanes) == 0
    chunk = n // num_workers

    buckets = jnp.linspace(-1.0, 1.0, _NUM_BUCKETS).astype(jnp.float32)
    x_flat = x.reshape(n)

    mesh = plsc.VectorSubcoreMesh(core_axis_name="c", subcore_axis_name="s")

    @functools.partial(
        pl.kernel,
        mesh=mesh,
        out_type=(
            jax.ShapeDtypeStruct((n,), jnp.int32),
            jax.ShapeDtypeStruct((n,), jnp.float32),
        ),
        scratch_types=[
            pltpu.VMEM((chunk,), jnp.float32),
            pltpu.VMEM((chunk,), jnp.int32),
            pltpu.VMEM((chunk,), jnp.float32),
            pltpu.VMEM((_NUM_BUCKETS,), jnp.float32),
        ],
    )
    def _quantize(x_hbm, b_hbm, idx_hbm, val_hbm, x_v, idx_v, val_v, b_v):
        wid = lax.axis_index("s") * num_cores + lax.axis_index("c")
        base = wid * chunk
        pltpu.sync_copy(b_hbm, b_v)
        pltpu.sync_copy(x_hbm.at[pl.ds(base, chunk)], x_v)
        b_vec = b_v[...]

        @plsc.parallel_loop(0, chunk, step=lanes, unroll=8)
        def _loop(off):
            v = x_v[pl.ds(off, lanes)]
            v = jnp.minimum(jnp.maximum(v, -1.0), 1.0)
            t = v * 7.5 + 8.0
            q = t.astype(jnp.int32)
            idx_v[pl.ds(off, lanes)] = q
            val_v[pl.ds(off, lanes)] = lax.gather(
                b_vec,
                q[:, None],
                dimension_numbers=lax.GatherDimensionNumbers(
                    offset_dims=(), collapsed_slice_dims=(0,),
                    start_index_map=(0,)),
                slice_sizes=(1,),
                mode=lax.GatherScatterMode.PROMISE_IN_BOUNDS,
            )

        pltpu.sync_copy(idx_v, idx_hbm.at[pl.ds(base, chunk)])
        pltpu.sync_copy(val_v, val_hbm.at[pl.ds(base, chunk)])

    idx, vals = _quantize(x_flat, buckets)
    return idx.reshape(orig_shape), vals.reshape(orig_shape)


# speedup vs baseline: 179.3968x; 1.0705x over previous
import functools

import jax
import jax.numpy as jnp
from jax import lax
from jax.experimental import pallas as pl
from jax.experimental.pallas import tpu as pltpu
from jax.experimental.pallas import tpu_sc as plsc

_NUM_BUCKETS = 16


def kernel(x):
    batch, rows, cols = x.shape
    info = plsc.get_sparse_core_info()
    num_cores, num_subcores, lanes = info.num_cores, info.num_subcores, info.num_lanes
    num_workers = num_cores * num_subcores
    blocks_per_batch = num_workers // batch
    row_blk = rows // blocks_per_batch
    col_groups = cols // lanes

    buckets = jnp.linspace(-1.0, 1.0, _NUM_BUCKETS).astype(jnp.float32)

    mesh = plsc.VectorSubcoreMesh(core_axis_name="c", subcore_axis_name="s")

    @functools.partial(
        pl.kernel,
        mesh=mesh,
        out_type=(
            jax.ShapeDtypeStruct((batch, rows, cols), jnp.int32),
            jax.ShapeDtypeStruct((batch, rows, cols), jnp.float32),
        ),
        scratch_types=[
            pltpu.VMEM((row_blk, cols), jnp.float32),
            pltpu.VMEM((row_blk, cols), jnp.int32),
            pltpu.VMEM((row_blk, cols), jnp.float32),
            pltpu.VMEM((_NUM_BUCKETS,), jnp.float32),
        ],
    )
    def _quantize(x_hbm, b_hbm, idx_hbm, val_hbm, x_v, idx_v, val_v, b_v):
        wid = lax.axis_index("s") * num_cores + lax.axis_index("c")
        b = wid // blocks_per_batch
        r0 = (wid % blocks_per_batch) * row_blk
        pltpu.sync_copy(b_hbm, b_v)
        pltpu.sync_copy(x_hbm.at[b, pl.ds(r0, row_blk), :], x_v)
        b_vec = b_v[...]
        dnums = lax.GatherDimensionNumbers(
            offset_dims=(), collapsed_slice_dims=(0,), start_index_map=(0,))

        @plsc.parallel_loop(0, row_blk, step=1, unroll=4)
        def _loop(r):
            for c in range(col_groups):
                v = x_v[r, pl.ds(c * lanes, lanes)]
                v = jnp.minimum(jnp.maximum(v, -1.0), 1.0)
                t = v * 7.5 + 8.0
                q = t.astype(jnp.int32)
                idx_v[r, pl.ds(c * lanes, lanes)] = q
                val_v[r, pl.ds(c * lanes, lanes)] = lax.gather(
                    b_vec, q[:, None], dimension_numbers=dnums,
                    slice_sizes=(1,),
                    mode=lax.GatherScatterMode.PROMISE_IN_BOUNDS,
                )

        pltpu.sync_copy(idx_v, idx_hbm.at[b, pl.ds(r0, row_blk), :])
        pltpu.sync_copy(val_v, val_hbm.at[b, pl.ds(r0, row_blk), :])

    return _quantize(x, buckets)


# speedup vs baseline: 179.4102x; 1.0001x over previous
import functools

import jax
import jax.numpy as jnp
from jax import lax
from jax.experimental import pallas as pl
from jax.experimental.pallas import tpu as pltpu
from jax.experimental.pallas import tpu_sc as plsc

_NUM_BUCKETS = 16


def kernel(x):
    batch, rows, cols = x.shape
    info = plsc.get_sparse_core_info()
    num_cores, num_subcores, lanes = info.num_cores, info.num_subcores, info.num_lanes
    num_workers = num_cores * num_subcores
    blocks_per_batch = num_workers // batch
    row_blk = rows // blocks_per_batch
    col_groups = cols // lanes

    buckets = jnp.linspace(-1.0, 1.0, _NUM_BUCKETS).astype(jnp.float32)

    mesh = plsc.VectorSubcoreMesh(core_axis_name="c", subcore_axis_name="s")

    @functools.partial(
        pl.kernel,
        mesh=mesh,
        out_type=(
            jax.ShapeDtypeStruct((batch, rows, cols), jnp.int32),
            jax.ShapeDtypeStruct((batch, rows, cols), jnp.float32),
        ),
        scratch_types=[
            pltpu.VMEM((row_blk, cols), jnp.float32),
            pltpu.VMEM((row_blk, cols), jnp.int32),
            pltpu.VMEM((row_blk, cols), jnp.float32),
            pltpu.VMEM((_NUM_BUCKETS,), jnp.float32),
        ],
        compiler_params=pltpu.CompilerParams(use_tc_tiling_on_sc=True),
    )
    def _quantize(x_hbm, b_hbm, idx_hbm, val_hbm, x_v, idx_v, val_v, b_v):
        wid = lax.axis_index("s") * num_cores + lax.axis_index("c")
        b = wid // blocks_per_batch
        r0 = (wid % blocks_per_batch) * row_blk
        pltpu.sync_copy(b_hbm, b_v)
        pltpu.sync_copy(x_hbm.at[b, pl.ds(r0, row_blk), :], x_v)
        b_vec = b_v[...]
        dnums = lax.GatherDimensionNumbers(
            offset_dims=(), collapsed_slice_dims=(0,), start_index_map=(0,))

        @plsc.parallel_loop(0, row_blk, step=1, unroll=4)
        def _loop(r):
            for c in range(col_groups):
                v = x_v[r, pl.ds(c * lanes, lanes)]
                v = jnp.minimum(jnp.maximum(v, -1.0), 1.0)
                t = v * 7.5 + 8.0
                q = t.astype(jnp.int32)
                idx_v[r, pl.ds(c * lanes, lanes)] = q
                val_v[r, pl.ds(c * lanes, lanes)] = lax.gather(
                    b_vec, q[:, None], dimension_numbers=dnums,
                    slice_sizes=(1,),
                    mode=lax.GatherScatterMode.PROMISE_IN_BOUNDS,
                )

        pltpu.sync_copy(idx_v, idx_hbm.at[b, pl.ds(r0, row_blk), :])
        pltpu.sync_copy(val_v, val_hbm.at[b, pl.ds(r0, row_blk), :])

    return _quantize(x, buckets)
